# baseline (speedup 1.0000x reference)
import dataclasses
import functools

import jax
import jax.numpy as jnp
from jax import lax
from jax.experimental import pallas as pl
from jax.experimental.pallas import tpu as pltpu
from jax.experimental.pallas import tpu_sc as plsc

N_E = 1000000
N_R = 1000
DIM = 64
B = 16384

NC = 2
NS = 16
NW = NC * NS
B_PER_W = B // NW
G = 128
NCH = B_PER_W // G
LANES = 16
W2 = 2 * DIM


def _sc_score_kernel(hrow_hbm, rrow_hbm, trow_hbm, hbase_hbm, rbase_hbm,
                     tbase_hbm, e2_hbm, r2_hbm, out_hbm,
                     idx_h, idx_r, idx_t, h_v, r_v, t_v, out_v,
                     hb_s, rb_s, tb_s, sem0, sem1):
    wid = lax.axis_index("s") * NC + lax.axis_index("c")

    pltpu.sync_copy(hrow_hbm.at[wid], idx_h)
    pltpu.sync_copy(rrow_hbm.at[wid], idx_r)
    pltpu.sync_copy(trow_hbm.at[wid], idx_t)
    pltpu.sync_copy(hbase_hbm.at[wid], hb_s)
    pltpu.sync_copy(rbase_hbm.at[wid], rb_s)
    pltpu.sync_copy(tbase_hbm.at[wid], tb_s)

    sems = (sem0, sem1)

    def fire(j):
        db, sem = j % 2, sems[j % 2]
        return [
            pltpu.async_copy(e2_hbm.at[idx_h.at[j]], h_v.at[db], sem),
            pltpu.async_copy(r2_hbm.at[idx_r.at[j]], r_v.at[db], sem),
            pltpu.async_copy(e2_hbm.at[idx_t.at[j]], t_v.at[db], sem),
        ]

    lane = lax.iota(jnp.int32, LANES)
    pend = fire(0)
    for j in range(NCH):
        nxt = fire(j + 1) if j + 1 < NCH else []
        for c in pend:
            c.wait()
        pend = nxt
        db = j % 2

        @pl.loop(0, G, step=LANES)
        def _(i0, j=j, db=db):
            row0 = j * G + i0
            bhv = hb_s[pl.ds(row0, LANES)]
            brv = rb_s[pl.ds(row0, LANES)]
            btv = tb_s[pl.ds(row0, LANES)]
            outv = jnp.zeros((LANES,), jnp.float32)
            for i in range(LANES):
                bh = bhv[i]
                br = brv[i]
                bt = btv[i]
                acc = jnp.zeros((LANES,), jnp.float32)
                for c in range(DIM // LANES):
                    o = c * LANES
                    hv = h_v[db, i0 + i, pl.ds(bh + o, LANES)]
                    rv = r_v[db, i0 + i, pl.ds(br + o, LANES)]
                    tv = t_v[db, i0 + i, pl.ds(bt + o, LANES)]
                    acc = acc + jnp.abs(hv + rv - tv)
                outv = jnp.where(lane == i, 100.0 - jnp.sum(acc), outv)
            out_v[pl.ds(j * G + i0, LANES)] = outv

    pltpu.sync_copy(out_v, out_hbm.at[pl.ds(wid * B_PER_W, B_PER_W)])


@jax.jit
def kernel(heads, rels, tails, E_table, R_table):
    heads = heads.astype(jnp.int32)
    rels = rels.astype(jnp.int32)
    tails = tails.astype(jnp.int32)
    e2 = E_table.reshape(N_E // 2, W2)
    r2 = R_table.reshape(N_R // 2, W2)

    def rows_bases(idx):
        return (jnp.reshape(idx >> 1, (NW, NCH, G)),
                jnp.reshape((idx & 1) << 6, (NW, B_PER_W)))

    hrow, hbase = rows_bases(heads)
    rrow, rbase = rows_bases(rels)
    trow, tbase = rows_bases(tails)

    cp = pltpu.CompilerParams()
    for fld, val in (("needs_layout_passes", False),):
        if fld in pltpu.CompilerParams.__dataclass_fields__:
            cp = dataclasses.replace(cp, **{fld: val})
    mesh = plsc.VectorSubcoreMesh(core_axis_name="c", subcore_axis_name="s")
    run = pl.kernel(
        _sc_score_kernel,
        out_type=jax.ShapeDtypeStruct((B,), jnp.float32),
        mesh=mesh,
        compiler_params=cp,
        scratch_types=[
            pltpu.VMEM((NCH, G), jnp.int32),
            pltpu.VMEM((NCH, G), jnp.int32),
            pltpu.VMEM((NCH, G), jnp.int32),
            pltpu.VMEM((2, G, W2), jnp.float32),
            pltpu.VMEM((2, G, W2), jnp.float32),
            pltpu.VMEM((2, G, W2), jnp.float32),
            pltpu.VMEM((B_PER_W,), jnp.float32),
            pltpu.VMEM((B_PER_W,), jnp.int32),
            pltpu.VMEM((B_PER_W,), jnp.int32),
            pltpu.VMEM((B_PER_W,), jnp.int32),
            pltpu.SemaphoreType.DMA,
            pltpu.SemaphoreType.DMA,
        ],
    )
    return run(hrow, rrow, trow, hbase, rbase, tbase, e2, r2)
